# baseline (speedup 1.0000x reference)
import functools

import jax
import jax.numpy as jnp
from jax.experimental import pallas as pl
from jax.experimental.pallas import tpu as pltpu

_BF16 = jnp.bfloat16

_T_OFF = {0: (-1, 0), 1: (0, 1)}


def _e1_body(a_ref, w_ref, bias_ref, o_ref):
    acc = jnp.dot(a_ref[...], w_ref[...], preferred_element_type=jnp.float32)
    acc = jnp.maximum(acc + bias_ref[0, :64][None, :], 0.0)
    o_ref[...] = acc.astype(o_ref.dtype)


def _e1_call(x, w1, b1):
    B, H, W, _ = x.shape
    Wo = W // 2
    xp = jnp.pad(x.astype(_BF16), ((0, 0), (3, 3), (3, 3), (0, 0)))
    halves = []
    for q in (0, 1):
        taps = [jax.lax.slice(xp, (0, di, q + dj, 0),
                              (B, di + H, q + dj + 2 * Wo - 1, 2),
                              (1, 1, 2, 1))
                for di in range(7) for dj in range(7)]
        halves.append(jnp.stack(taps, axis=3))
    cols = jnp.stack(halves, axis=2).reshape(B * H * W, 98)
    M = B * H * W
    TM = min(4096, M)
    out = pl.pallas_call(
        _e1_body,
        out_shape=jax.ShapeDtypeStruct((M, 64), _BF16),
        grid_spec=pltpu.PrefetchScalarGridSpec(
            num_scalar_prefetch=0,
            grid=(M // TM,),
            in_specs=[
                pl.BlockSpec((TM, 98), lambda i: (i, 0)),
                pl.BlockSpec((98, 64), lambda i: (0, 0)),
                pl.BlockSpec((1, 128), lambda i: (0, 0)),
            ],
            out_specs=pl.BlockSpec((TM, 64), lambda i: (i, 0)),
            scratch_shapes=[],
        ),
        compiler_params=pltpu.CompilerParams(
            dimension_semantics=("parallel",)),
    )(cols, w1, b1)
    return out.reshape(B, H, 2, Wo, 64)


def _enc_body(xs_ref, w_ref, bias_ref, o_ref, s0, s1, *, TH, Wo, C, N,
              mid_split):
    r = pl.program_id(1)

    @pl.when(r == 0)
    def _():
        if mid_split:
            ev, od = xs_ref[0, :, 0], xs_ref[0, :, 1]
        else:
            ev, od = xs_ref[0, 0], xs_ref[0, 1]
        s0[...] = jnp.pad(od, ((1, 1), (1, 0), (0, 0)))
        s1[...] = jnp.pad(ev, ((1, 1), (0, 1), (0, 0)))

    W2 = Wo + 1
    xt0 = s0[pl.ds(2 * r * TH, 2 * TH + 2), :, :]
    xt1 = s1[pl.ds(2 * r * TH, 2 * TH + 2), :, :]
    re0 = xt0.reshape(TH + 1, 2, W2, C)
    re1 = xt1.reshape(TH + 1, 2, W2, C)
    taps = []
    for di in range(4):
        p, a = di & 1, di // 2
        for dj in range(4):
            q, b = dj & 1, dj // 2
            re = re0 if q == 0 else re1
            taps.append(re[a:a + TH, p, b:b + Wo, :])
    a_mat = jnp.concatenate(taps, axis=2).reshape(TH * Wo, 16 * C)
    acc = jnp.dot(a_mat, w_ref[...], preferred_element_type=jnp.float32)
    acc = jnp.maximum(acc + bias_ref[0][None, :], 0.0)
    o_ref[0] = acc.reshape(TH, Wo, N).astype(o_ref.dtype)


def _enc_call(x, w, b, N, TH, mid_split=False):
    if mid_split:
        B, Hi, _, Wo, C = x.shape
        xs = x
        in_block = (1, Hi, 2, Wo, C)
    else:
        B, Hi, Wi, C = x.shape
        Wo = Wi // 2
        xs = x.reshape(B, Hi, Wo, 2, C).transpose(0, 3, 1, 2, 4)
        in_block = (1, 2, Hi, Wo, C)
    Ho = Hi // 2
    TH = min(TH, Ho)
    return pl.pallas_call(
        functools.partial(_enc_body, TH=TH, Wo=Wo, C=C, N=N,
                          mid_split=mid_split),
        out_shape=jax.ShapeDtypeStruct((B, Ho, Wo, N), _BF16),
        grid_spec=pltpu.PrefetchScalarGridSpec(
            num_scalar_prefetch=0,
            grid=(B, Ho // TH),
            in_specs=[
                pl.BlockSpec(in_block, lambda b, r: (b, 0, 0, 0, 0)),
                pl.BlockSpec((16 * C, N), lambda b, r: (0, 0)),
                pl.BlockSpec((1, N), lambda b, r: (0, 0)),
            ],
            out_specs=pl.BlockSpec((1, TH, Wo, N), lambda b, r: (b, r, 0, 0)),
            scratch_shapes=[
                pltpu.VMEM((Hi + 2, Wo + 1, C), _BF16),
                pltpu.VMEM((Hi + 2, Wo + 1, C), _BF16),
            ],
        ),
        compiler_params=pltpu.CompilerParams(
            dimension_semantics=("parallel", "arbitrary")),
    )(xs, w, b)


def _dec_body(x_ref, w_ref, bias_ref, o_ref, sx, *, TH, Wi, C, N, NO):
    r = pl.program_id(1)

    @pl.when(r == 0)
    def _():
        sx[...] = jnp.pad(x_ref[0], ((1, 1), (1, 1), (0, 0)))

    row0 = r * TH
    xt = sx[pl.ds(row0, TH + 2), :, :]
    phases = []
    for pr in (0, 1):
        for pc in (0, 1):
            wins = []
            for oa in _T_OFF[pr]:
                for ob in _T_OFF[pc]:
                    wins.append(xt[1 + oa:1 + oa + TH, 1 + ob:1 + ob + Wi, :])
            a = jnp.concatenate(wins, axis=2).reshape(TH * Wi, 4 * C)
            acc = jnp.dot(a, w_ref[2 * pr + pc],
                          preferred_element_type=jnp.float32)
            acc = jnp.maximum(acc + bias_ref[0][None, :], 0.0)
            phases.append(acc[:, :N].reshape(TH, Wi, N).astype(o_ref.dtype))
    row_a = jnp.concatenate([phases[0], phases[1]], axis=2)
    row_b = jnp.concatenate([phases[2], phases[3]], axis=2)
    yy = jnp.stack([row_a, row_b], axis=1)
    o_ref[0] = yy


def _dec_fused_body(x_ref, w_ref, bias_ref, v_ref, o_ref, sx,
                    *, TH, Wi, C, N, NO):
    r = pl.program_id(1)

    @pl.when(r == 0)
    def _():
        sx[...] = jnp.pad(x_ref[0], ((1, 1), (1, 1), (0, 0)))

    row0 = r * TH
    xt = sx[pl.ds(row0, TH + 2), :, :]
    phases = []
    for pr in (0, 1):
        for pc in (0, 1):
            wins = []
            for oa in _T_OFF[pr]:
                for ob in _T_OFF[pc]:
                    wins.append(xt[1 + oa:1 + oa + TH, 1 + ob:1 + ob + Wi, :])
            a = jnp.concatenate(wins, axis=2).reshape(TH * Wi, 4 * C)
            acc = jnp.dot(a, w_ref[2 * pr + pc],
                          preferred_element_type=jnp.float32)
            acc = jnp.maximum(acc + bias_ref[0][None, :], 0.0)
            act = acc[:, :N].astype(_BF16)
            s = jnp.dot(act, v_ref[...], preferred_element_type=jnp.float32)
            phases.append(s[:, :NO].reshape(TH, Wi, NO).astype(o_ref.dtype))
    row_a = jnp.concatenate([phases[0], phases[1]], axis=2)
    row_b = jnp.concatenate([phases[2], phases[3]], axis=2)
    yy = jnp.stack([row_a, row_b], axis=1)
    o_ref[0] = yy


def _dec_call(x, w, b, N, TH, v=None, NO=None):
    B, Hi, Wi, C = x.shape
    TH = min(TH, Hi)
    Np = w.shape[2]
    NO = N if v is None else NO
    in_specs = [
        pl.BlockSpec((1, Hi, Wi, C), lambda b, r: (b, 0, 0, 0)),
        pl.BlockSpec((4, 4 * C, Np), lambda b, r: (0, 0, 0)),
        pl.BlockSpec((1, Np), lambda b, r: (0, 0)),
    ]
    args = [x, w, b]
    if v is None:
        body = functools.partial(_dec_body, TH=TH, Wi=Wi, C=C, N=N, NO=NO)
    else:
        body = functools.partial(_dec_fused_body, TH=TH, Wi=Wi, C=C, N=N,
                                 NO=NO)
        in_specs.append(pl.BlockSpec((N, 128), lambda b, r: (0, 0)))
        args.append(v)
    out = pl.pallas_call(
        body,
        out_shape=jax.ShapeDtypeStruct((B, Hi, 2, Wi, 2 * NO), _BF16),
        grid_spec=pltpu.PrefetchScalarGridSpec(
            num_scalar_prefetch=0,
            grid=(B, Hi // TH),
            in_specs=in_specs,
            out_specs=pl.BlockSpec((1, TH, 2, Wi, 2 * NO),
                                   lambda b, r: (b, r, 0, 0, 0)),
            scratch_shapes=[pltpu.VMEM((Hi + 2, Wi + 2, C), _BF16)],
        ),
        compiler_params=pltpu.CompilerParams(
            dimension_semantics=("parallel", "arbitrary")),
    )(*args)
    return out.reshape(B, 2 * Hi, 2 * Wi, NO)


def _d4b_body(s_ref, bias_ref, o_ref, sx, *, TH, W):
    r = pl.program_id(1)

    @pl.when(r == 0)
    def _():
        sx[...] = jnp.pad(s_ref[0], ((0, 0), (3, 5), (3, 3)))

    row0 = r * TH
    xt = sx[:, pl.ds(row0, TH + 8), :]
    acc = jnp.full((TH, W), bias_ref[0, 0], jnp.float32)
    for di in range(7):
        for dj in range(7):
            acc = acc + xt[di * 7 + dj, di:di + TH, dj:dj + W]
    o_ref[0] = jnp.tanh(acc)


def _d4_call(s1, b4):
    B, H, W, _ = s1.shape
    TH = min(32, H)
    s2 = jnp.transpose(s1, (0, 3, 1, 2))
    return pl.pallas_call(
        functools.partial(_d4b_body, TH=TH, W=W),
        out_shape=jax.ShapeDtypeStruct((B, H, W), jnp.float32),
        grid_spec=pltpu.PrefetchScalarGridSpec(
            num_scalar_prefetch=0,
            grid=(B, H // TH),
            in_specs=[
                pl.BlockSpec((1, 49, H, W), lambda b, r: (b, 0, 0, 0)),
                pl.BlockSpec((1, 128), lambda b, r: (0, 0)),
            ],
            out_specs=pl.BlockSpec((1, TH, W), lambda b, r: (b, r, 0)),
            scratch_shapes=[pltpu.VMEM((49, H + 8, W + 6), _BF16)],
        ),
        compiler_params=pltpu.CompilerParams(
            dimension_semantics=("parallel", "arbitrary")),
    )(s2, b4)


@jax.jit
def _run(frame_a, frame_b, e1_w, e1_b, e2_w, e2_b, e3_w, e3_b, e4_w, e4_b,
         d1_w, d1_b, d2_w, d2_b, d3_w, d3_b, d4_w, d4_b):
    B = frame_a.shape[0]
    x01 = jnp.stack([frame_a[:, 0], frame_b[:, 0]], axis=-1)
    w1 = e1_w[0, :98, :64]
    vA = jnp.pad(d4_w[0, :3136, 0].reshape(49, 64).T, ((0, 0), (0, 79)))

    x = _e1_call(x01, w1, e1_b)
    x = _enc_call(x, e2_w[0], e2_b, 128, 8, mid_split=True)
    x = _enc_call(x, e3_w[0], e3_b, 256, 16)
    x = _enc_call(x, e4_w[0], e4_b, 512, 32)
    x = _dec_call(x, d1_w, d1_b, 256, 32)
    x = _dec_call(x, d2_w, d2_b, 128, 16)
    s1 = _dec_call(x, d3_w, d3_b, 64, 16, v=vA, NO=49)
    y = _d4_call(s1, d4_b)
    return y[:, None]


def kernel(frame_a, frame_b, e1_w, e1_b, e2_w, e2_b, e3_w, e3_b, e4_w, e4_b,
           d1_w, d1_b, d2_w, d2_b, d3_w, d3_b, d4_w, d4_b):
    return _run(frame_a, frame_b, e1_w, e1_b, e2_w, e2_b, e3_w, e3_b,
                e4_w, e4_b, d1_w, d1_b, d2_w, d2_b, d3_w, d3_b, d4_w, d4_b)


# speedup vs baseline: 2.2206x; 2.2206x over previous
import functools

import jax
import jax.numpy as jnp
from jax.experimental import pallas as pl
from jax.experimental.pallas import tpu as pltpu

_BF16 = jnp.bfloat16

_T_OFF = {0: (-1, 0), 1: (0, 1)}


def _e1_body(a_ref, w_ref, bias_ref, o_ref):
    acc = jnp.dot(a_ref[...], w_ref[...], preferred_element_type=jnp.float32)
    acc = jnp.maximum(acc + bias_ref[0, :64][None, :], 0.0)
    o_ref[...] = acc.astype(o_ref.dtype)


def _e1_call(x, w1, b1):
    B, H, W, _ = x.shape
    xp = jnp.pad(x.astype(_BF16), ((0, 0), (3, 3), (3, 3), (0, 0)))
    taps = [xp[:, di:di + H, dj:dj + W, :]
            for di in range(7) for dj in range(7)]
    cols = jnp.stack(taps, axis=3).reshape(B * H * W, 98)
    M = B * H * W
    TM = min(4096, M)
    out = pl.pallas_call(
        _e1_body,
        out_shape=jax.ShapeDtypeStruct((M, 64), _BF16),
        grid_spec=pltpu.PrefetchScalarGridSpec(
            num_scalar_prefetch=0,
            grid=(M // TM,),
            in_specs=[
                pl.BlockSpec((TM, 98), lambda i: (i, 0)),
                pl.BlockSpec((98, 64), lambda i: (0, 0)),
                pl.BlockSpec((1, 128), lambda i: (0, 0)),
            ],
            out_specs=pl.BlockSpec((TM, 64), lambda i: (i, 0)),
            scratch_shapes=[],
        ),
        compiler_params=pltpu.CompilerParams(
            dimension_semantics=("parallel",)),
    )(cols, w1, b1)
    return out.reshape(B, H, W, 64)


def _enc_body(xs_ref, w_ref, bias_ref, o_ref, s0, s1, *, TH, Wo, C, N,
              mid_split):
    r = pl.program_id(1)

    @pl.when(r == 0)
    def _():
        if mid_split:
            ev, od = xs_ref[0, :, 0], xs_ref[0, :, 1]
        else:
            ev, od = xs_ref[0, 0], xs_ref[0, 1]
        s0[...] = jnp.pad(od, ((1, 1), (1, 0), (0, 0)))
        s1[...] = jnp.pad(ev, ((1, 1), (0, 1), (0, 0)))

    W2 = Wo + 1
    xt0 = s0[pl.ds(2 * r * TH, 2 * TH + 2), :, :]
    xt1 = s1[pl.ds(2 * r * TH, 2 * TH + 2), :, :]
    re0 = xt0.reshape(TH + 1, 2, W2, C)
    re1 = xt1.reshape(TH + 1, 2, W2, C)
    taps = []
    for di in range(4):
        p, a = di & 1, di // 2
        for dj in range(4):
            q, b = dj & 1, dj // 2
            re = re0 if q == 0 else re1
            taps.append(re[a:a + TH, p, b:b + Wo, :])
    a_mat = jnp.concatenate(taps, axis=2).reshape(TH * Wo, 16 * C)
    acc = jnp.dot(a_mat, w_ref[...], preferred_element_type=jnp.float32)
    acc = jnp.maximum(acc + bias_ref[0][None, :], 0.0)
    o_ref[0] = acc.reshape(TH, Wo, N).astype(o_ref.dtype)


def _enc_call(x, w, b, N, TH, mid_split=False):
    if mid_split:
        B, Hi, _, Wo, C = x.shape
        xs = x
        in_block = (1, Hi, 2, Wo, C)
    else:
        B, Hi, Wi, C = x.shape
        Wo = Wi // 2
        xs = x.reshape(B, Hi, Wo, 2, C).transpose(0, 3, 1, 2, 4)
        in_block = (1, 2, Hi, Wo, C)
    Ho = Hi // 2
    TH = min(TH, Ho)
    return pl.pallas_call(
        functools.partial(_enc_body, TH=TH, Wo=Wo, C=C, N=N,
                          mid_split=mid_split),
        out_shape=jax.ShapeDtypeStruct((B, Ho, Wo, N), _BF16),
        grid_spec=pltpu.PrefetchScalarGridSpec(
            num_scalar_prefetch=0,
            grid=(B, Ho // TH),
            in_specs=[
                pl.BlockSpec(in_block, lambda b, r: (b, 0, 0, 0, 0)),
                pl.BlockSpec((16 * C, N), lambda b, r: (0, 0)),
                pl.BlockSpec((1, N), lambda b, r: (0, 0)),
            ],
            out_specs=pl.BlockSpec((1, TH, Wo, N), lambda b, r: (b, r, 0, 0)),
            scratch_shapes=[
                pltpu.VMEM((Hi + 2, Wo + 1, C), _BF16),
                pltpu.VMEM((Hi + 2, Wo + 1, C), _BF16),
            ],
        ),
        compiler_params=pltpu.CompilerParams(
            dimension_semantics=("parallel", "arbitrary")),
    )(xs, w, b)


def _dec_body(x_ref, w_ref, bias_ref, o_ref, sx, *, TH, Wi, C, N, NO):
    r = pl.program_id(1)

    @pl.when(r == 0)
    def _():
        sx[...] = jnp.pad(x_ref[0], ((1, 1), (1, 1), (0, 0)))

    row0 = r * TH
    xt = sx[pl.ds(row0, TH + 2), :, :]
    phases = []
    for pr in (0, 1):
        for pc in (0, 1):
            wins = []
            for oa in _T_OFF[pr]:
                for ob in _T_OFF[pc]:
                    wins.append(xt[1 + oa:1 + oa + TH, 1 + ob:1 + ob + Wi, :])
            a = jnp.concatenate(wins, axis=2).reshape(TH * Wi, 4 * C)
            acc = jnp.dot(a, w_ref[2 * pr + pc],
                          preferred_element_type=jnp.float32)
            acc = jnp.maximum(acc + bias_ref[0][None, :], 0.0)
            phases.append(acc[:, :N].reshape(TH, Wi, N).astype(o_ref.dtype))
    row_a = jnp.concatenate([phases[0], phases[1]], axis=2)
    row_b = jnp.concatenate([phases[2], phases[3]], axis=2)
    yy = jnp.stack([row_a, row_b], axis=1)
    o_ref[0] = yy


def _dec_fused_body(x_ref, w_ref, bias_ref, v_ref, o_ref, sx,
                    *, TH, Wi, C, N, NO):
    r = pl.program_id(1)

    @pl.when(r == 0)
    def _():
        sx[...] = jnp.pad(x_ref[0], ((1, 1), (1, 1), (0, 0)))

    row0 = r * TH
    xt = sx[pl.ds(row0, TH + 2), :, :]
    phases = []
    for pr in (0, 1):
        for pc in (0, 1):
            wins = []
            for oa in _T_OFF[pr]:
                for ob in _T_OFF[pc]:
                    wins.append(xt[1 + oa:1 + oa + TH, 1 + ob:1 + ob + Wi, :])
            a = jnp.concatenate(wins, axis=2).reshape(TH * Wi, 4 * C)
            acc = jnp.dot(a, w_ref[2 * pr + pc],
                          preferred_element_type=jnp.float32)
            acc = jnp.maximum(acc + bias_ref[0][None, :], 0.0)
            act = acc[:, :N].astype(_BF16)
            s = jnp.dot(act, v_ref[...], preferred_element_type=jnp.float32)
            phases.append(s[:, :NO].reshape(TH, Wi, NO).astype(o_ref.dtype))
    row_a = jnp.concatenate([phases[0], phases[1]], axis=2)
    row_b = jnp.concatenate([phases[2], phases[3]], axis=2)
    yy = jnp.stack([row_a, row_b], axis=1)
    o_ref[0] = yy


def _dec_call(x, w, b, N, TH, v=None, NO=None):
    B, Hi, Wi, C = x.shape
    TH = min(TH, Hi)
    Np = w.shape[2]
    NO = N if v is None else NO
    in_specs = [
        pl.BlockSpec((1, Hi, Wi, C), lambda b, r: (b, 0, 0, 0)),
        pl.BlockSpec((4, 4 * C, Np), lambda b, r: (0, 0, 0)),
        pl.BlockSpec((1, Np), lambda b, r: (0, 0)),
    ]
    args = [x, w, b]
    if v is None:
        body = functools.partial(_dec_body, TH=TH, Wi=Wi, C=C, N=N, NO=NO)
    else:
        body = functools.partial(_dec_fused_body, TH=TH, Wi=Wi, C=C, N=N,
                                 NO=NO)
        in_specs.append(pl.BlockSpec((N, 128), lambda b, r: (0, 0)))
        args.append(v)
    out = pl.pallas_call(
        body,
        out_shape=jax.ShapeDtypeStruct((B, Hi, 2, Wi, 2 * NO), _BF16),
        grid_spec=pltpu.PrefetchScalarGridSpec(
            num_scalar_prefetch=0,
            grid=(B, Hi // TH),
            in_specs=in_specs,
            out_specs=pl.BlockSpec((1, TH, 2, Wi, 2 * NO),
                                   lambda b, r: (b, r, 0, 0, 0)),
            scratch_shapes=[pltpu.VMEM((Hi + 2, Wi + 2, C), _BF16)],
        ),
        compiler_params=pltpu.CompilerParams(
            dimension_semantics=("parallel", "arbitrary")),
    )(*args)
    return out.reshape(B, 2 * Hi, 2 * Wi, NO)


def _d4b_body(s_ref, bias_ref, o_ref, sx, *, TH, W):
    r = pl.program_id(1)

    @pl.when(r == 0)
    def _():
        sx[...] = jnp.pad(s_ref[0], ((0, 0), (3, 5), (3, 3)))

    row0 = r * TH
    xt = sx[:, pl.ds(row0, TH + 8), :]
    acc = jnp.full((TH, W), bias_ref[0, 0], jnp.float32)
    for di in range(7):
        for dj in range(7):
            acc = acc + xt[di * 7 + dj, di:di + TH, dj:dj + W]
    o_ref[0] = jnp.tanh(acc)


def _d4_call(s1, b4):
    B, H, W, _ = s1.shape
    TH = min(32, H)
    s2 = jnp.transpose(s1, (0, 3, 1, 2))
    return pl.pallas_call(
        functools.partial(_d4b_body, TH=TH, W=W),
        out_shape=jax.ShapeDtypeStruct((B, H, W), jnp.float32),
        grid_spec=pltpu.PrefetchScalarGridSpec(
            num_scalar_prefetch=0,
            grid=(B, H // TH),
            in_specs=[
                pl.BlockSpec((1, 49, H, W), lambda b, r: (b, 0, 0, 0)),
                pl.BlockSpec((1, 128), lambda b, r: (0, 0)),
            ],
            out_specs=pl.BlockSpec((1, TH, W), lambda b, r: (b, r, 0)),
            scratch_shapes=[pltpu.VMEM((49, H + 8, W + 6), _BF16)],
        ),
        compiler_params=pltpu.CompilerParams(
            dimension_semantics=("parallel", "arbitrary")),
    )(s2, b4)


@jax.jit
def _run(frame_a, frame_b, e1_w, e1_b, e2_w, e2_b, e3_w, e3_b, e4_w, e4_b,
         d1_w, d1_b, d2_w, d2_b, d3_w, d3_b, d4_w, d4_b):
    B = frame_a.shape[0]
    x01 = jnp.stack([frame_a[:, 0], frame_b[:, 0]], axis=-1)
    w1 = e1_w[0, :98, :64]
    vA = jnp.pad(d4_w[0, :3136, 0].reshape(49, 64).T, ((0, 0), (0, 79)))

    x = _e1_call(x01, w1, e1_b)
    x = _enc_call(x, e2_w[0], e2_b, 128, 8)
    x = _enc_call(x, e3_w[0], e3_b, 256, 16)
    x = _enc_call(x, e4_w[0], e4_b, 512, 32)
    x = _dec_call(x, d1_w, d1_b, 256, 32)
    x = _dec_call(x, d2_w, d2_b, 128, 16)
    s1 = _dec_call(x, d3_w, d3_b, 64, 16, v=vA, NO=49)
    y = _d4_call(s1, d4_b)
    return y[:, None]


def kernel(frame_a, frame_b, e1_w, e1_b, e2_w, e2_b, e3_w, e3_b, e4_w, e4_b,
           d1_w, d1_b, d2_w, d2_b, d3_w, d3_b, d4_w, d4_b):
    return _run(frame_a, frame_b, e1_w, e1_b, e2_w, e2_b, e3_w, e3_b,
                e4_w, e4_b, d1_w, d1_b, d2_w, d2_b, d3_w, d3_b, d4_w, d4_b)


# speedup vs baseline: 2.3567x; 1.0613x over previous
import functools

import jax
import jax.numpy as jnp
from jax.experimental import pallas as pl
from jax.experimental.pallas import tpu as pltpu

_BF16 = jnp.bfloat16

_T_OFF = {0: (-1, 0), 1: (0, 1)}


def _e1_body(a_ref, w_ref, bias_ref, o_ref):
    acc = jnp.dot(a_ref[...], w_ref[...], preferred_element_type=jnp.float32)
    acc = jnp.maximum(acc + bias_ref[0, :64][None, :], 0.0)
    o_ref[...] = acc.astype(o_ref.dtype)


def _e1_call(x, w1, b1):
    B, H, W, _ = x.shape
    xp = jnp.pad(x.astype(_BF16), ((0, 0), (3, 3), (3, 3), (0, 0)))
    taps = [xp[:, di:di + H, dj:dj + W, :]
            for di in range(7) for dj in range(7)]
    cols = jnp.stack(taps, axis=3).reshape(B * H * W, 98)
    M = B * H * W
    TM = min(8192, M)
    out = pl.pallas_call(
        _e1_body,
        out_shape=jax.ShapeDtypeStruct((M, 64), _BF16),
        grid_spec=pltpu.PrefetchScalarGridSpec(
            num_scalar_prefetch=0,
            grid=(M // TM,),
            in_specs=[
                pl.BlockSpec((TM, 98), lambda i: (i, 0)),
                pl.BlockSpec((98, 64), lambda i: (0, 0)),
                pl.BlockSpec((1, 128), lambda i: (0, 0)),
            ],
            out_specs=pl.BlockSpec((TM, 64), lambda i: (i, 0)),
            scratch_shapes=[],
        ),
        compiler_params=pltpu.CompilerParams(
            dimension_semantics=("parallel",)),
    )(cols, w1, b1)
    return out.reshape(B, H, W, 64)


def _enc_body(xs_ref, w_ref, bias_ref, o_ref, s0, s1, *, TH, Wo, C, N,
              mid_split):
    r = pl.program_id(1)

    @pl.when(r == 0)
    def _():
        if mid_split:
            ev, od = xs_ref[0, :, 0], xs_ref[0, :, 1]
        else:
            ev, od = xs_ref[0, 0], xs_ref[0, 1]
        s0[...] = jnp.pad(od, ((1, 1), (1, 0), (0, 0)))
        s1[...] = jnp.pad(ev, ((1, 1), (0, 1), (0, 0)))

    W2 = Wo + 1
    xt0 = s0[pl.ds(2 * r * TH, 2 * TH + 2), :, :]
    xt1 = s1[pl.ds(2 * r * TH, 2 * TH + 2), :, :]
    re0 = xt0.reshape(TH + 1, 2, W2, C)
    re1 = xt1.reshape(TH + 1, 2, W2, C)
    taps = []
    for di in range(4):
        p, a = di & 1, di // 2
        for dj in range(4):
            q, b = dj & 1, dj // 2
            re = re0 if q == 0 else re1
            taps.append(re[a:a + TH, p, b:b + Wo, :])
    a_mat = jnp.concatenate(taps, axis=2).reshape(TH * Wo, 16 * C)
    acc = jnp.dot(a_mat, w_ref[...], preferred_element_type=jnp.float32)
    acc = jnp.maximum(acc + bias_ref[0][None, :], 0.0)
    o_ref[0] = acc.reshape(TH, Wo, N).astype(o_ref.dtype)


def _enc_call(x, w, b, N, TH, mid_split=False):
    if mid_split:
        B, Hi, _, Wo, C = x.shape
        xs = x
        in_block = (1, Hi, 2, Wo, C)
    else:
        B, Hi, Wi, C = x.shape
        Wo = Wi // 2
        xs = x.reshape(B, Hi, Wo, 2, C).transpose(0, 3, 1, 2, 4)
        in_block = (1, 2, Hi, Wo, C)
    Ho = Hi // 2
    TH = min(TH, Ho)
    return pl.pallas_call(
        functools.partial(_enc_body, TH=TH, Wo=Wo, C=C, N=N,
                          mid_split=mid_split),
        out_shape=jax.ShapeDtypeStruct((B, Ho, Wo, N), _BF16),
        grid_spec=pltpu.PrefetchScalarGridSpec(
            num_scalar_prefetch=0,
            grid=(B, Ho // TH),
            in_specs=[
                pl.BlockSpec(in_block, lambda b, r: (b, 0, 0, 0, 0)),
                pl.BlockSpec((16 * C, N), lambda b, r: (0, 0)),
                pl.BlockSpec((1, N), lambda b, r: (0, 0)),
            ],
            out_specs=pl.BlockSpec((1, TH, Wo, N), lambda b, r: (b, r, 0, 0)),
            scratch_shapes=[
                pltpu.VMEM((Hi + 2, Wo + 1, C), _BF16),
                pltpu.VMEM((Hi + 2, Wo + 1, C), _BF16),
            ],
        ),
        compiler_params=pltpu.CompilerParams(
            dimension_semantics=("parallel", "arbitrary")),
    )(xs, w, b)


def _dec_body(x_ref, w_ref, bias_ref, o_ref, sx, *, TH, Wi, C, N, NO):
    r = pl.program_id(1)

    @pl.when(r == 0)
    def _():
        sx[...] = jnp.pad(x_ref[0], ((1, 1), (1, 1), (0, 0)))

    row0 = r * TH
    xt = sx[pl.ds(row0, TH + 2), :, :]
    phases = []
    for pr in (0, 1):
        for pc in (0, 1):
            wins = []
            for oa in _T_OFF[pr]:
                for ob in _T_OFF[pc]:
                    wins.append(xt[1 + oa:1 + oa + TH, 1 + ob:1 + ob + Wi, :])
            a = jnp.concatenate(wins, axis=2).reshape(TH * Wi, 4 * C)
            acc = jnp.dot(a, w_ref[2 * pr + pc],
                          preferred_element_type=jnp.float32)
            acc = jnp.maximum(acc + bias_ref[0][None, :], 0.0)
            phases.append(acc[:, :N].reshape(TH, Wi, N).astype(o_ref.dtype))
    row_a = jnp.concatenate([phases[0], phases[1]], axis=2)
    row_b = jnp.concatenate([phases[2], phases[3]], axis=2)
    yy = jnp.stack([row_a, row_b], axis=1)
    o_ref[0] = yy


def _dec_fused_body(x_ref, w_ref, bias_ref, v_ref, o_ref, sx,
                    *, TH, Wi, C, N, NO):
    r = pl.program_id(1)

    @pl.when(r == 0)
    def _():
        sx[...] = jnp.pad(x_ref[0], ((1, 1), (1, 1), (0, 0)))

    row0 = r * TH
    xt = sx[pl.ds(row0, TH + 2), :, :]
    phases = []
    for pr in (0, 1):
        for pc in (0, 1):
            wins = []
            for oa in _T_OFF[pr]:
                for ob in _T_OFF[pc]:
                    wins.append(xt[1 + oa:1 + oa + TH, 1 + ob:1 + ob + Wi, :])
            a = jnp.concatenate(wins, axis=2).reshape(TH * Wi, 4 * C)
            acc = jnp.dot(a, w_ref[2 * pr + pc],
                          preferred_element_type=jnp.float32)
            acc = jnp.maximum(acc + bias_ref[0][None, :], 0.0)
            act = acc[:, :N].astype(_BF16)
            s = jnp.dot(act, v_ref[...], preferred_element_type=jnp.float32)
            phases.append(s[:, :NO].reshape(TH, Wi, NO).astype(o_ref.dtype))
    row_a = jnp.concatenate([phases[0], phases[1]], axis=2)
    row_b = jnp.concatenate([phases[2], phases[3]], axis=2)
    yy = jnp.stack([row_a, row_b], axis=1)
    o_ref[0] = yy


def _dec_call(x, w, b, N, TH, v=None, NO=None):
    B, Hi, Wi, C = x.shape
    TH = min(TH, Hi)
    Np = w.shape[2]
    NO = N if v is None else NO
    in_specs = [
        pl.BlockSpec((1, Hi, Wi, C), lambda b, r: (b, 0, 0, 0)),
        pl.BlockSpec((4, 4 * C, Np), lambda b, r: (0, 0, 0)),
        pl.BlockSpec((1, Np), lambda b, r: (0, 0)),
    ]
    args = [x, w, b]
    if v is None:
        body = functools.partial(_dec_body, TH=TH, Wi=Wi, C=C, N=N, NO=NO)
    else:
        body = functools.partial(_dec_fused_body, TH=TH, Wi=Wi, C=C, N=N,
                                 NO=NO)
        in_specs.append(pl.BlockSpec((N, 128), lambda b, r: (0, 0)))
        args.append(v)
    out = pl.pallas_call(
        body,
        out_shape=jax.ShapeDtypeStruct((B, Hi, 2, Wi, 2 * NO), _BF16),
        grid_spec=pltpu.PrefetchScalarGridSpec(
            num_scalar_prefetch=0,
            grid=(B, Hi // TH),
            in_specs=in_specs,
            out_specs=pl.BlockSpec((1, TH, 2, Wi, 2 * NO),
                                   lambda b, r: (b, r, 0, 0, 0)),
            scratch_shapes=[pltpu.VMEM((Hi + 2, Wi + 2, C), _BF16)],
        ),
        compiler_params=pltpu.CompilerParams(
            dimension_semantics=("parallel", "arbitrary")),
    )(*args)
    return out.reshape(B, 2 * Hi, 2 * Wi, NO)


def _d4b_body(s_ref, bias_ref, o_ref, sx, *, TH, W):
    r = pl.program_id(1)

    @pl.when(r == 0)
    def _():
        sx[...] = jnp.pad(s_ref[0], ((0, 0), (3, 5), (3, 3)))

    row0 = r * TH
    xt = sx[:, pl.ds(row0, TH + 8), :]
    acc = jnp.full((TH, W), bias_ref[0, 0], jnp.float32)
    for di in range(7):
        for dj in range(7):
            acc = acc + xt[di * 7 + dj, di:di + TH, dj:dj + W]
    o_ref[0] = jnp.tanh(acc)


def _d4_call(s1, b4):
    B, H, W, _ = s1.shape
    TH = min(64, H)
    s2 = jnp.transpose(s1, (0, 3, 1, 2))
    return pl.pallas_call(
        functools.partial(_d4b_body, TH=TH, W=W),
        out_shape=jax.ShapeDtypeStruct((B, H, W), jnp.float32),
        grid_spec=pltpu.PrefetchScalarGridSpec(
            num_scalar_prefetch=0,
            grid=(B, H // TH),
            in_specs=[
                pl.BlockSpec((1, 49, H, W), lambda b, r: (b, 0, 0, 0)),
                pl.BlockSpec((1, 128), lambda b, r: (0, 0)),
            ],
            out_specs=pl.BlockSpec((1, TH, W), lambda b, r: (b, r, 0)),
            scratch_shapes=[pltpu.VMEM((49, H + 8, W + 6), _BF16)],
        ),
        compiler_params=pltpu.CompilerParams(
            dimension_semantics=("parallel", "arbitrary")),
    )(s2, b4)


@jax.jit
def _run(frame_a, frame_b, e1_w, e1_b, e2_w, e2_b, e3_w, e3_b, e4_w, e4_b,
         d1_w, d1_b, d2_w, d2_b, d3_w, d3_b, d4_w, d4_b):
    B = frame_a.shape[0]
    x01 = jnp.stack([frame_a[:, 0], frame_b[:, 0]], axis=-1)
    w1 = e1_w[0, :98, :64]
    vA = jnp.pad(d4_w[0, :3136, 0].reshape(49, 64).T, ((0, 0), (0, 79)))

    x = _e1_call(x01, w1, e1_b)
    x = _enc_call(x, e2_w[0], e2_b, 128, 16)
    x = _enc_call(x, e3_w[0], e3_b, 256, 32)
    x = _enc_call(x, e4_w[0], e4_b, 512, 32)
    x = _dec_call(x, d1_w, d1_b, 256, 32)
    x = _dec_call(x, d2_w, d2_b, 128, 32)
    s1 = _dec_call(x, d3_w, d3_b, 64, 32, v=vA, NO=49)
    y = _d4_call(s1, d4_b)
    return y[:, None]


def kernel(frame_a, frame_b, e1_w, e1_b, e2_w, e2_b, e3_w, e3_b, e4_w, e4_b,
           d1_w, d1_b, d2_w, d2_b, d3_w, d3_b, d4_w, d4_b):
    return _run(frame_a, frame_b, e1_w, e1_b, e2_w, e2_b, e3_w, e3_b,
                e4_w, e4_b, d1_w, d1_b, d2_w, d2_b, d3_w, d3_b, d4_w, d4_b)
